# baseline (speedup 1.0000x reference)
import functools

import jax
import jax.numpy as jnp
from jax import lax
from jax.experimental import pallas as pl
from jax.experimental.pallas import tpu as pltpu
from jax.experimental.pallas import tpu_sc as plsc

N_NODES = 10000
N_EDGES = 320000
D_IN = 128
D_HID = 128
D_OUT = 40
D_PAD = 48

NC = 2
NS = 16
NW = NC * NS
BATCH = 125
NB = 80
N_ACC = N_NODES
ROWS_PT = N_NODES // NS

_MESH = dict(core_axis_name="c", subcore_axis_name="s")


def _prop_body(D, wave, idxc, src_hbm, dst_hbm, xs_hbm, z_hbm, out_hbm,
               idx_s, idx_d, rows_v, acc_sh, sem, gsems):
    c = lax.axis_index("c")
    s = lax.axis_index("s")
    wid = c * NS + s
    pltpu.sync_copy(z_hbm.at[pl.ds(s * ROWS_PT, ROWS_PT)],
                    acc_sh.at[pl.ds(s * ROWS_PT, ROWS_PT)])
    plsc.subcore_barrier()

    def chunk(k, carry):
        base = wid * NB + k * idxc
        pltpu.sync_copy(src_hbm.at[pl.ds(base, idxc)], idx_s)
        pltpu.sync_copy(dst_hbm.at[pl.ds(base, idxc)], idx_d)

        def step(i, carry2):
            j0 = i * wave
            gathers = [
                pltpu.async_copy(xs_hbm.at[idx_s.at[j0 + w]], rows_v.at[w],
                                 gsems.at[w])
                for w in range(wave)
            ]
            for w in range(wave):
                gathers[w].wait()
                pltpu.sync_copy(rows_v.at[w], acc_sh.at[idx_d.at[j0 + w]],
                                add=True)
            return carry2

        return lax.fori_loop(0, idxc // wave, step, carry)

    lax.fori_loop(0, NB // idxc, chunk, 0)
    plsc.subcore_barrier()
    pltpu.sync_copy(acc_sh.at[pl.ds(s * ROWS_PT, ROWS_PT)],
                    out_hbm.at[pl.ds(c * N_NODES + s * ROWS_PT, ROWS_PT)])


_SC_PARAMS = pltpu.CompilerParams(use_tc_tiling_on_sc=False)


def _make_prop(D, wave, idxc):
    return pl.kernel(
        functools.partial(_prop_body, D, wave, idxc),
        out_type=jax.ShapeDtypeStruct((NC * N_NODES, D), jnp.float32),
        mesh=plsc.VectorSubcoreMesh(**_MESH),
        compiler_params=_SC_PARAMS,
        scratch_types=[
            pltpu.VMEM((idxc, BATCH), jnp.int32),
            pltpu.VMEM((idxc, BATCH), jnp.int32),
            pltpu.VMEM((wave, BATCH, D), jnp.float32),
            pltpu.VMEM_SHARED((N_ACC, D), jnp.float32),
            pltpu.SemaphoreType.DMA,
            pltpu.SemaphoreType.DMA((wave,)),
        ],
    )


def _deg_body(dst_hbm, ones_hbm, z_hbm, out_hbm, idx_d, ones_v, acc_sh, sem):
    c = lax.axis_index("c")
    s = lax.axis_index("s")
    wid = c * NS + s
    pltpu.sync_copy(z_hbm.at[pl.ds(s * ROWS_PT, ROWS_PT)],
                    acc_sh.at[pl.ds(s * ROWS_PT, ROWS_PT)])
    pltpu.sync_copy(ones_hbm, ones_v)
    pltpu.sync_copy(dst_hbm.at[pl.ds(wid * NB, NB)], idx_d)
    plsc.subcore_barrier()

    def step(j, carry):
        pltpu.sync_copy(ones_v, acc_sh.at[idx_d.at[j]], add=True)
        return carry

    lax.fori_loop(0, NB, step, 0)
    plsc.subcore_barrier()
    pltpu.sync_copy(acc_sh.at[pl.ds(s * ROWS_PT, ROWS_PT)],
                    out_hbm.at[pl.ds(c * N_NODES + s * ROWS_PT, ROWS_PT)])


_deg_kernel = pl.kernel(
    _deg_body,
    out_type=jax.ShapeDtypeStruct((NC * N_NODES, 16), jnp.float32),
    mesh=plsc.VectorSubcoreMesh(**_MESH),
    compiler_params=_SC_PARAMS,
    scratch_types=[
        pltpu.VMEM((NB, BATCH), jnp.int32),
        pltpu.VMEM((BATCH, 16), jnp.float32),
        pltpu.VMEM_SHARED((N_ACC, 16), jnp.float32),
        pltpu.SemaphoreType.DMA,
    ],
)


BM = 1000


def _dinv_of(d0, d1):
    cnt = d0[:, 0:1] + d1[:, 0:1]
    return lax.rsqrt(cnt + 1.0)


def _tc1_body(x, w1, d0, d1, xs_out):
    dinv = _dinv_of(d0, d1)
    xs_out[...] = jnp.dot(x[...], w1[...],
                          preferred_element_type=jnp.float32) * dinv


def _tc2_body(a0, a1, xs, d0, d1, b1, w2, h_out, ys_out):
    dinv = _dinv_of(d0, d1)
    h = (a0[...] + a1[...] + xs[...]) * dinv + b1[...]
    h_out[...] = h
    y = jnp.maximum(h, 0.0)
    ys_out[...] = jnp.dot(y, w2[...],
                          preferred_element_type=jnp.float32) * dinv


def _tc3_body(a0, a1, ys, d0, d1, b2, logp_out):
    dinv = _dinv_of(d0, d1)
    o = ((a0[...] + a1[...] + ys[...]) * dinv)[:, :D_OUT] + b2[...]
    m = jnp.max(o, axis=1, keepdims=True)
    e = jnp.exp(o - m)
    lse = jnp.log(jnp.sum(e, axis=1, keepdims=True))
    logp_out[...] = o - m - lse


def _row_spec(d):
    return pl.BlockSpec((BM, d), lambda i: (i, 0))


def _half_spec(d, half):
    return pl.BlockSpec((BM, d), lambda i, h=half: (i + h * _GRID, 0))


def _full_spec(r, c):
    return pl.BlockSpec((r, c), lambda i: (0, 0))


_GRID = N_NODES // BM

_tc1 = pl.pallas_call(
    _tc1_body,
    grid=(_GRID,),
    in_specs=[_row_spec(D_IN), _full_spec(D_IN, D_HID),
              _half_spec(16, 0), _half_spec(16, 1)],
    out_specs=_row_spec(D_HID),
    out_shape=jax.ShapeDtypeStruct((N_NODES, D_HID), jnp.float32),
)

_tc2 = pl.pallas_call(
    _tc2_body,
    grid=(_GRID,),
    in_specs=[_half_spec(D_HID, 0), _half_spec(D_HID, 1), _row_spec(D_HID),
              _half_spec(16, 0), _half_spec(16, 1),
              _full_spec(1, D_HID), _full_spec(D_HID, D_PAD)],
    out_specs=[_row_spec(D_HID), _row_spec(D_PAD)],
    out_shape=[jax.ShapeDtypeStruct((N_NODES, D_HID), jnp.float32),
               jax.ShapeDtypeStruct((N_NODES, D_PAD), jnp.float32)],
)

_tc3 = pl.pallas_call(
    _tc3_body,
    grid=(_GRID,),
    in_specs=[_half_spec(D_PAD, 0), _half_spec(D_PAD, 1), _row_spec(D_PAD),
              _half_spec(16, 0), _half_spec(16, 1), _full_spec(1, D_OUT)],
    out_specs=_row_spec(D_OUT),
    out_shape=jax.ShapeDtypeStruct((N_NODES, D_OUT), jnp.float32),
)


def kernel(x, edge_index, W1, b1, W2, b2):
    src = edge_index[0].astype(jnp.int32)
    dst = edge_index[1].astype(jnp.int32)
    src3 = src.reshape(NW * NB, BATCH)
    dst3 = dst.reshape(NW * NB, BATCH)

    z128 = jnp.zeros((N_NODES, D_HID), jnp.float32)
    z48 = jnp.zeros((N_NODES, D_PAD), jnp.float32)
    z16 = jnp.zeros((N_NODES, 16), jnp.float32)
    ones = jnp.ones((BATCH, 16), jnp.float32)

    deg = _deg_kernel(dst3, ones, z16)

    xs = _tc1(x, W1, deg, deg)

    acc1 = _make_prop(D_HID, 2, 40)(src3, dst3, xs, z128)

    W2p = jnp.zeros((D_HID, D_PAD), jnp.float32).at[:, :D_OUT].set(W2)
    h, ys = _tc2(acc1, acc1, xs, deg, deg, b1.reshape(1, D_HID), W2p)

    acc2 = _make_prop(D_PAD, 8, 8)(src3, dst3, ys, z48)

    logp = _tc3(acc2, acc2, ys, deg, deg, b2.reshape(1, D_OUT))
    return (h, logp)


# speedup vs baseline: 1.0158x; 1.0158x over previous
import functools

import jax
import jax.numpy as jnp
from jax import lax
from jax.experimental import pallas as pl
from jax.experimental.pallas import tpu as pltpu
from jax.experimental.pallas import tpu_sc as plsc

N_NODES = 10000
N_EDGES = 320000
D_IN = 128
D_HID = 128
D_OUT = 40
D_PAD = 48

NC = 2
NS = 16
NW = NC * NS
BATCH = 125
NB = 80
N_ACC = N_NODES
ROWS_PT = N_NODES // NS

_MESH = dict(core_axis_name="c", subcore_axis_name="s")


def _prop_body(D, wave, idxc, src_hbm, dst_hbm, xs_hbm, z_hbm, out_hbm,
               idx_s, idx_d, rows_v, acc_sh, sem, gsems):
    c = lax.axis_index("c")
    s = lax.axis_index("s")
    wid = c * NS + s
    pltpu.sync_copy(z_hbm.at[pl.ds(s * ROWS_PT, ROWS_PT)],
                    acc_sh.at[pl.ds(s * ROWS_PT, ROWS_PT)])
    plsc.subcore_barrier()

    def chunk(k, carry):
        base = wid * NB + k * idxc
        pltpu.sync_copy(src_hbm.at[pl.ds(base, idxc)], idx_s)
        pltpu.sync_copy(dst_hbm.at[pl.ds(base, idxc)], idx_d)

        def step(i, carry2):
            j0 = i * wave
            gathers = [
                pltpu.async_copy(xs_hbm.at[idx_s.at[j0 + w]], rows_v.at[w],
                                 gsems.at[w])
                for w in range(wave)
            ]
            for w in range(wave):
                gathers[w].wait()
                pltpu.sync_copy(rows_v.at[w], acc_sh.at[idx_d.at[j0 + w]],
                                add=True)
            return carry2

        return lax.fori_loop(0, idxc // wave, step, carry)

    lax.fori_loop(0, NB // idxc, chunk, 0)
    plsc.subcore_barrier()
    pltpu.sync_copy(acc_sh.at[pl.ds(s * ROWS_PT, ROWS_PT)],
                    out_hbm.at[pl.ds(c * N_NODES + s * ROWS_PT, ROWS_PT)])


_SC_PARAMS = pltpu.CompilerParams(use_tc_tiling_on_sc=False)


def _make_prop(D, wave, idxc):
    return pl.kernel(
        functools.partial(_prop_body, D, wave, idxc),
        out_type=jax.ShapeDtypeStruct((NC * N_NODES, D), jnp.float32),
        mesh=plsc.VectorSubcoreMesh(**_MESH),
        compiler_params=_SC_PARAMS,
        scratch_types=[
            pltpu.VMEM((idxc, BATCH), jnp.int32),
            pltpu.VMEM((idxc, BATCH), jnp.int32),
            pltpu.VMEM((wave, BATCH, D), jnp.float32),
            pltpu.VMEM_SHARED((N_ACC, D), jnp.float32),
            pltpu.SemaphoreType.DMA,
            pltpu.SemaphoreType.DMA((wave,)),
        ],
    )


def _deg_body(dst_hbm, ones_hbm, z_hbm, out_hbm, idx_d, ones_v, acc_sh, sem):
    c = lax.axis_index("c")
    s = lax.axis_index("s")
    wid = c * NS + s
    pltpu.sync_copy(z_hbm.at[pl.ds(s * ROWS_PT, ROWS_PT)],
                    acc_sh.at[pl.ds(s * ROWS_PT, ROWS_PT)])
    pltpu.sync_copy(ones_hbm, ones_v)
    pltpu.sync_copy(dst_hbm.at[pl.ds(wid * NB, NB)], idx_d)
    plsc.subcore_barrier()

    def step(j, carry):
        pltpu.sync_copy(ones_v, acc_sh.at[idx_d.at[j]], add=True)
        return carry

    lax.fori_loop(0, NB, step, 0)
    plsc.subcore_barrier()
    pltpu.sync_copy(acc_sh.at[pl.ds(s * ROWS_PT, ROWS_PT)],
                    out_hbm.at[pl.ds(c * N_NODES + s * ROWS_PT, ROWS_PT)])


_deg_kernel = pl.kernel(
    _deg_body,
    out_type=jax.ShapeDtypeStruct((NC * N_NODES, 16), jnp.float32),
    mesh=plsc.VectorSubcoreMesh(**_MESH),
    compiler_params=_SC_PARAMS,
    scratch_types=[
        pltpu.VMEM((NB, BATCH), jnp.int32),
        pltpu.VMEM((BATCH, 16), jnp.float32),
        pltpu.VMEM_SHARED((N_ACC, 16), jnp.float32),
        pltpu.SemaphoreType.DMA,
    ],
)


BM = 1000


def _dinv_of(d0, d1):
    cnt = d0[:, 0:1] + d1[:, 0:1]
    return lax.rsqrt(cnt + 1.0)


def _tc1_body(x, w1, d0, d1, xs_out):
    dinv = _dinv_of(d0, d1)
    xs_out[...] = jnp.dot(x[...], w1[...],
                          preferred_element_type=jnp.float32) * dinv


def _tc2_body(a0, a1, xs, d0, d1, b1, w2, h_out, ys_out):
    dinv = _dinv_of(d0, d1)
    h = (a0[...] + a1[...] + xs[...]) * dinv + b1[...]
    h_out[...] = h
    y = jnp.maximum(h, 0.0)
    ys_out[...] = jnp.dot(y, w2[...],
                          preferred_element_type=jnp.float32) * dinv


def _tc3_body(a0, a1, ys, d0, d1, b2, logp_out):
    dinv = _dinv_of(d0, d1)
    o = ((a0[...] + a1[...] + ys[...]) * dinv)[:, :D_OUT] + b2[...]
    m = jnp.max(o, axis=1, keepdims=True)
    e = jnp.exp(o - m)
    lse = jnp.log(jnp.sum(e, axis=1, keepdims=True))
    logp_out[...] = o - m - lse


def _row_spec(d):
    return pl.BlockSpec((BM, d), lambda i: (i, 0))


def _half_spec(d, half):
    return pl.BlockSpec((BM, d), lambda i, h=half: (i + h * _GRID, 0))


def _full_spec(r, c):
    return pl.BlockSpec((r, c), lambda i: (0, 0))


_GRID = N_NODES // BM

_tc1 = pl.pallas_call(
    _tc1_body,
    grid=(_GRID,),
    in_specs=[_row_spec(D_IN), _full_spec(D_IN, D_HID),
              _half_spec(16, 0), _half_spec(16, 1)],
    out_specs=_row_spec(D_HID),
    out_shape=jax.ShapeDtypeStruct((N_NODES, D_HID), jnp.float32),
)

_tc2 = pl.pallas_call(
    _tc2_body,
    grid=(_GRID,),
    in_specs=[_half_spec(D_HID, 0), _half_spec(D_HID, 1), _row_spec(D_HID),
              _half_spec(16, 0), _half_spec(16, 1),
              _full_spec(1, D_HID), _full_spec(D_HID, D_PAD)],
    out_specs=[_row_spec(D_HID), _row_spec(D_PAD)],
    out_shape=[jax.ShapeDtypeStruct((N_NODES, D_HID), jnp.float32),
               jax.ShapeDtypeStruct((N_NODES, D_PAD), jnp.float32)],
)

_tc3 = pl.pallas_call(
    _tc3_body,
    grid=(_GRID,),
    in_specs=[_half_spec(D_PAD, 0), _half_spec(D_PAD, 1), _row_spec(D_PAD),
              _half_spec(16, 0), _half_spec(16, 1), _full_spec(1, D_OUT)],
    out_specs=_row_spec(D_OUT),
    out_shape=jax.ShapeDtypeStruct((N_NODES, D_OUT), jnp.float32),
)


def kernel(x, edge_index, W1, b1, W2, b2):
    src = edge_index[0].astype(jnp.int32)
    dst = edge_index[1].astype(jnp.int32)
    src3 = src.reshape(NW * NB, BATCH)
    dst3 = dst.reshape(NW * NB, BATCH)

    z128 = jnp.zeros((N_NODES, D_HID), jnp.float32)
    z48 = jnp.zeros((N_NODES, D_PAD), jnp.float32)
    z16 = jnp.zeros((N_NODES, 16), jnp.float32)
    ones = jnp.ones((BATCH, 16), jnp.float32)

    deg = _deg_kernel(dst3, ones, z16)

    xs = _tc1(x, W1, deg, deg)

    acc1 = _make_prop(D_HID, 2, 40)(src3, dst3, xs, z128)

    W2p = jnp.zeros((D_HID, D_PAD), jnp.float32).at[:, :D_OUT].set(W2)
    h, ys = _tc2(acc1, acc1, xs, deg, deg, b1.reshape(1, D_HID), W2p)

    acc2 = _make_prop(D_PAD, 4, 80)(src3, dst3, ys, z48)

    logp = _tc3(acc2, acc2, ys, deg, deg, b2.reshape(1, D_OUT))
    return (h, logp)
